# baseline (speedup 1.0000x reference)
import functools

import jax
import jax.numpy as jnp
from jax import lax
from jax.experimental import pallas as pl
from jax.experimental.pallas import tpu as pltpu
from jax.experimental.pallas import tpu_sc as plsc


def kernel(x, pos_embed):
    seq_len = x.shape[0]
    d_model = pos_embed.shape[1]
    pe3 = pos_embed[:seq_len].reshape(seq_len, 1, d_model)
    mesh = plsc.VectorSubcoreMesh(core_axis_name="c", subcore_axis_name="s")

    @functools.partial(
        pl.kernel,
        mesh=mesh,
        out_type=jax.ShapeDtypeStruct((seq_len, 1, d_model), pos_embed.dtype),
    )
    def sc_lookup(pe_hbm, out_hbm):
        wid = lax.axis_index("s") * jax.lax.axis_size("c") + lax.axis_index("c")

        @pl.when(wid == 0)
        def _():
            pltpu.sync_copy(pe_hbm, out_hbm)

    return sc_lookup(pe3)


# speedup vs baseline: 1.1799x; 1.1799x over previous
import functools

import jax
import jax.numpy as jnp
from jax import lax
from jax.experimental import pallas as pl
from jax.experimental.pallas import tpu as pltpu
from jax.experimental.pallas import tpu_sc as plsc


def kernel(x, pos_embed):
    seq_len = x.shape[0]
    d_model = pos_embed.shape[1]
    pe3 = pos_embed[:seq_len].reshape(seq_len, 1, d_model)
    mesh = plsc.ScalarSubcoreMesh(axis_name="c", num_cores=1)

    @functools.partial(
        pl.kernel,
        mesh=mesh,
        out_type=jax.ShapeDtypeStruct((seq_len, 1, d_model), pos_embed.dtype),
    )
    def sc_lookup(pe_hbm, out_hbm):
        pltpu.sync_copy(pe_hbm, out_hbm)

    return sc_lookup(pe3)


# speedup vs baseline: 14.4564x; 12.2523x over previous
import jax
import jax.numpy as jnp
from jax.experimental import pallas as pl


def _lookup_body(pe_ref, out_ref):
    out_ref[:, 0, :] = pe_ref[...]


def kernel(x, pos_embed):
    seq_len = x.shape[0]
    d_model = pos_embed.shape[1]
    return pl.pallas_call(
        _lookup_body,
        out_shape=jax.ShapeDtypeStruct((seq_len, 1, d_model), pos_embed.dtype),
    )(pos_embed[:seq_len])
